# baseline (speedup 1.0000x reference)
import jax
import jax.numpy as jnp
from jax import lax
from jax.experimental import pallas as pl
from jax.experimental.pallas import tpu as pltpu
from jax.experimental.pallas import tpu_sc as plsc

N = 100_000
E = 1_600_000
D_IN = 4
D_H = 64

NC = 2
NS = 16
E_PAD = 1_638_400
N_PAD = 102_400
NT_ROWS = N_PAD // NS

def _get_mesh():
  return plsc.VectorSubcoreMesh(
      core_axis_name="c", subcore_axis_name="s", num_cores=NC, num_subcores=NS)


def _zero_acc(zeros_hbm, acc, sid, nt=NT_ROWS):
  pltpu.sync_copy(zeros_hbm.at[pl.ds(sid * nt, nt)],
                  acc.at[pl.ds(sid * nt, nt)])


DSUB = 256
CSUB = 1
STEP_E = DSUB * CSUB


def _deg_body(dst1, zeros1, out0, out1, acc, i0, i1, i2, i3, i4, i5, i6, i7,
              ones, semi, sems):
  cid = lax.axis_index("c")
  sid = lax.axis_index("s")
  ixd = ((i0, i1), (i2, i3), (i4, i5), (i6, i7))
  for k in range(DSUB // 16):
    ones[pl.ds(16 * k, 16)] = jnp.full((16,), 1.0, jnp.float32)
  _zero_acc(zeros1, acc, sid)
  plsc.subcore_barrier()
  ept = E_PAD // (NC * NS)
  base = cid * (E_PAD // 2) + sid * ept
  nstep = ept // STEP_E

  def loadidx(slot, c):
    for j in range(CSUB):
      pltpu.async_copy(dst1.at[pl.ds(base + c * STEP_E + j * DSUB, DSUB)],
                       ixd[slot][j], semi)

  def iwait():
    for j in range(CSUB):
      pltpu.make_async_copy(dst1.at[pl.ds(0, DSUB)], ixd[0][j], semi).wait()

  def fire_s(slot):
    for j in range(CSUB):
      pltpu.async_copy(ones, acc.at[ixd[slot][j]], sems, add=True)

  def sdrain():
    for j in range(CSUB):
      pltpu.make_async_copy(zeros1.at[pl.ds(0, DSUB)], ones, sems).wait()

  loadidx(0, 0)
  loadidx(1, 1)

  def step4(q, carry):
    for k in range(4):
      c = 4 * q + k

      @pl.when(c >= 2)
      def _():
        sdrain()
      iwait()
      fire_s(k)

      @pl.when(c + 2 < nstep)
      def _():
        loadidx((k + 2) % 4, c + 2)
    return carry
  lax.fori_loop(0, nstep // 4, step4, 0)
  sdrain()
  sdrain()
  plsc.subcore_barrier()

  @pl.when(cid == 0)
  def _():
    pltpu.sync_copy(acc.at[pl.ds(sid * NT_ROWS, NT_ROWS)],
                    out0.at[pl.ds(sid * NT_ROWS, NT_ROWS)])

  @pl.when(cid == 1)
  def _():
    pltpu.sync_copy(acc.at[pl.ds(sid * NT_ROWS, NT_ROWS)],
                    out1.at[pl.ds(sid * NT_ROWS, NT_ROWS)])


def _deg_call(dst1, zeros1):
  return pl.kernel(
      _deg_body,
      out_type=[jax.ShapeDtypeStruct((N_PAD,), jnp.float32),
                jax.ShapeDtypeStruct((N_PAD,), jnp.float32)],
      mesh=_get_mesh(),
      compiler_params=pltpu.CompilerParams(use_tc_tiling_on_sc=False),
      scratch_types=(
          [pltpu.VMEM_SHARED((N_PAD,), jnp.float32)]
          + [pltpu.VMEM((DSUB,), jnp.int32) for _ in range(8)]
          + [pltpu.VMEM((DSUB,), jnp.float32),
             pltpu.SemaphoreType.DMA,
             pltpu.SemaphoreType.DMA]
      ),
  )(dst1, zeros1)


A1_W = 8
A2_W = 16
IXR = 8
RBR = 4


def _agg_stream(tbl, acc, src1, dst1, zrows, base, nstep, w,
                ixs, ixd, rbs, semi, semg, sems):
  def loadidx(slot, c):
    for j in range(CSUB):
      off = base + c * STEP_E + j * DSUB
      pltpu.async_copy(src1.at[pl.ds(off, DSUB)], ixs[slot][j], semi)
      pltpu.async_copy(dst1.at[pl.ds(off, DSUB)], ixd[slot][j], semi)

  def iwait():
    for j in range(2 * CSUB):
      pltpu.make_async_copy(src1.at[pl.ds(0, DSUB)], ixs[0][0], semi).wait()

  def fire_g(slot, rb):
    for j in range(CSUB):
      pltpu.async_copy(tbl.at[ixs[slot][j]],
                       rb.at[pl.ds(j * DSUB, DSUB)], semg)

  def fire_s(slot, rb):
    for j in range(CSUB):
      pltpu.async_copy(rb.at[pl.ds(j * DSUB, DSUB)],
                       acc.at[ixd[slot][j]], sems, add=True)

  def gdrain():
    pltpu.make_async_copy(zrows, rbs[0], semg).wait()

  def sdrain():
    pltpu.make_async_copy(zrows, rbs[0], sems).wait()

  loadidx(0, 0)
  loadidx(1, 1)

  def step8(q, carry):
    for k in range(IXR):
      c = IXR * q + k

      @pl.when(c >= 4)
      def _():
        sdrain()
      iwait()
      fire_g(k, rbs[k % RBR])

      @pl.when(c + 2 < nstep)
      def _():
        loadidx((k + 2) % IXR, c + 2)

      @pl.when(c >= 1)
      def _():
        gdrain()
        fire_s((k - 1) % IXR, rbs[(k - 1) % RBR])
    return carry
  lax.fori_loop(0, nstep // IXR, step8, 0)
  gdrain()
  fire_s((nstep - 1) % IXR, rbs[(nstep - 1) % RBR])
  for _ in range(4):
    sdrain()


def _agg_scratch(w):
  return ([pltpu.VMEM((DSUB,), jnp.int32) for _ in range(2 * CSUB * IXR)]
          + [pltpu.VMEM((STEP_E, w), jnp.float32) for _ in range(RBR)]
          + [pltpu.SemaphoreType.DMA] * 3)


def _slots(refs):
  n = CSUB * IXR
  ixs = tuple(tuple(refs[s * CSUB + j] for j in range(CSUB))
              for s in range(IXR))
  ixd = tuple(tuple(refs[n + s * CSUB + j] for j in range(CSUB))
              for s in range(IXR))
  rbs = tuple(refs[2 * n:2 * n + RBR])
  semi, semg, sems = refs[2 * n + RBR:2 * n + RBR + 3]
  return ixs, ixd, rbs, semi, semg, sems


def _agg1_body(src1, dst1, xs, zeros8, out0, out1, acc, xsp, *refs):
  cid = lax.axis_index("c")
  sid = lax.axis_index("s")
  ixs, ixd, rbs, semi, semg, sems = _slots(refs)
  _zero_acc(zeros8, acc, sid)
  pltpu.sync_copy(xs.at[pl.ds(sid * NT_ROWS, NT_ROWS)],
                  xsp.at[pl.ds(sid * NT_ROWS, NT_ROWS)])
  plsc.subcore_barrier()
  ept = E_PAD // (NC * NS)
  base = cid * (E_PAD // 2) + sid * ept
  _agg_stream(xsp, acc, src1, dst1, zeros8.at[pl.ds(0, STEP_E)],
              base, ept // STEP_E, A1_W, ixs, ixd, rbs, semi, semg, sems)
  plsc.subcore_barrier()

  @pl.when(cid == 0)
  def _():
    pltpu.sync_copy(acc.at[pl.ds(sid * NT_ROWS, NT_ROWS)],
                    out0.at[pl.ds(sid * NT_ROWS, NT_ROWS)])

  @pl.when(cid == 1)
  def _():
    pltpu.sync_copy(acc.at[pl.ds(sid * NT_ROWS, NT_ROWS)],
                    out1.at[pl.ds(sid * NT_ROWS, NT_ROWS)])


def _agg1_call(src1, dst1, xs, zeros8):
  return pl.kernel(
      _agg1_body,
      out_type=[jax.ShapeDtypeStruct((N_PAD, A1_W), jnp.float32),
                jax.ShapeDtypeStruct((N_PAD, A1_W), jnp.float32)],
      mesh=_get_mesh(),
      compiler_params=pltpu.CompilerParams(use_tc_tiling_on_sc=False),
      scratch_types=(
          [pltpu.VMEM_SHARED((N_PAD, A1_W), jnp.float32),
           pltpu.VMEM_SHARED((N_PAD, A1_W), jnp.float32)]
          + _agg_scratch(A1_W)
      ),
  )(src1, dst1, xs, zeros8)


def _agg2_quarter(src1, dst1, tbl, zeros16, out, acc, slots, sid):
  ixs, ixd, rbs, semi, semg, sems = slots
  _zero_acc(zeros16, acc, sid)
  plsc.subcore_barrier()
  ept = E_PAD // NS
  base = sid * ept
  _agg_stream(tbl, acc, src1, dst1, zeros16.at[pl.ds(0, STEP_E)],
              base, ept // STEP_E, A2_W, ixs, ixd, rbs, semi, semg, sems)
  plsc.subcore_barrier()
  pltpu.sync_copy(acc.at[pl.ds(sid * NT_ROWS, NT_ROWS)],
                  out.at[pl.ds(sid * NT_ROWS, NT_ROWS)])
  plsc.subcore_barrier()


def _agg2_body(src1, dst1, t0, t1, t2, t3, zeros16, o0, o1, o2, o3, acc,
               *refs):
  cid = lax.axis_index("c")
  sid = lax.axis_index("s")
  slots = _slots(refs)

  @pl.when(cid == 0)
  def _():
    _agg2_quarter(src1, dst1, t0, zeros16, o0, acc, slots, sid)
    _agg2_quarter(src1, dst1, t1, zeros16, o1, acc, slots, sid)

  @pl.when(cid == 1)
  def _():
    _agg2_quarter(src1, dst1, t2, zeros16, o2, acc, slots, sid)
    _agg2_quarter(src1, dst1, t3, zeros16, o3, acc, slots, sid)


def _agg2_call(src1, dst1, t0, t1, t2, t3, zeros16):
  sds = jax.ShapeDtypeStruct((N_PAD, A2_W), jnp.float32)
  return pl.kernel(
      _agg2_body,
      out_type=[sds, sds, sds, sds],
      mesh=_get_mesh(),
      compiler_params=pltpu.CompilerParams(use_tc_tiling_on_sc=False),
      scratch_types=(
          [pltpu.VMEM_SHARED((N_PAD, A2_W), jnp.float32)]
          + _agg_scratch(A2_W)
      ),
  )(src1, dst1, t0, t1, t2, t3, zeros16)


NB = 2000
GRID = N // NB


def _tca_body(dg0_ref, dg1_ref, x8_ref, dinv_ref, xs_ref):
  deg = dg0_ref[...] + dg1_ref[...] + 1.0
  dinv = lax.rsqrt(deg)
  dinv_ref[...] = dinv
  xs_ref[...] = x8_ref[...] * dinv


def _tca(dg0, dg1, x8):
  return pl.pallas_call(
      _tca_body,
      grid=(GRID,),
      in_specs=[
          pl.BlockSpec((NB, 1), lambda i: (i, 0)),
          pl.BlockSpec((NB, 1), lambda i: (i, 0)),
          pl.BlockSpec((NB, A1_W), lambda i: (i, 0)),
      ],
      out_specs=[
          pl.BlockSpec((NB, 1), lambda i: (i, 0)),
          pl.BlockSpec((NB, A1_W), lambda i: (i, 0)),
      ],
      out_shape=[
          jax.ShapeDtypeStruct((N, 1), jnp.float32),
          jax.ShapeDtypeStruct((N_PAD, A1_W), jnp.float32),
      ],
  )(dg0, dg1, x8)


def _layer_norm_relu(h, g, b):
  mu = jnp.mean(h, axis=1, keepdims=True)
  xc = h - mu
  var = jnp.mean(xc * xc, axis=1, keepdims=True)
  return jnp.maximum(xc * lax.rsqrt(var + 1e-5) * g + b, 0.0)


def _tcb_body(aggp0_ref, aggp1_ref, x8_ref, dinv_ref, w1_ref, p_ref,
              o0_ref, o1_ref, o2_ref, o3_ref):
  di = dinv_ref[...]
  agg = aggp0_ref[...] + aggp1_ref[...]
  pre = di * agg + (di * di) * x8_ref[...]
  h = jnp.dot(pre, w1_ref[...], preferred_element_type=jnp.float32)
  h = h + p_ref[0]
  h = _layer_norm_relu(h, p_ref[1], p_ref[2])
  h = h * di
  o0_ref[...] = h[:, 0:16]
  o1_ref[...] = h[:, 16:32]
  o2_ref[...] = h[:, 32:48]
  o3_ref[...] = h[:, 48:64]


def _tcb(aggp0, aggp1, x8, dinv, w1p, params):
  sds = jax.ShapeDtypeStruct((N, A2_W), jnp.float32)
  bs = pl.BlockSpec((NB, A2_W), lambda i: (i, 0))
  return pl.pallas_call(
      _tcb_body,
      grid=(GRID,),
      in_specs=[
          pl.BlockSpec((NB, A1_W), lambda i: (i, 0)),
          pl.BlockSpec((NB, A1_W), lambda i: (i, 0)),
          pl.BlockSpec((NB, A1_W), lambda i: (i, 0)),
          pl.BlockSpec((NB, 1), lambda i: (i, 0)),
          pl.BlockSpec((A1_W, D_H), lambda i: (0, 0)),
          pl.BlockSpec((8, D_H), lambda i: (0, 0)),
      ],
      out_specs=[bs, bs, bs, bs],
      out_shape=[sds, sds, sds, sds],
  )(aggp0, aggp1, x8, dinv, w1p, params)


def _tcc_body(a0, a1, a2, a3, h0, h1, h2, h3, dinv_ref, w2_ref, fw1_ref,
              fw2_ref, p_ref, out_ref):
  di = dinv_ref[...]
  w2 = w2_ref[...]
  z = jnp.zeros((NB, D_H), jnp.float32)
  for k, (a, h) in enumerate(((a0, h0), (a1, h1), (a2, h2), (a3, h3))):
    zk = di * (a[...] + h[...])
    z = z + jnp.dot(zk, w2[16 * k:16 * (k + 1), :],
                    preferred_element_type=jnp.float32)
  z = z + p_ref[0]
  z = _layer_norm_relu(z, p_ref[1], p_ref[2])
  z = jnp.maximum(
      jnp.dot(z, fw1_ref[...], preferred_element_type=jnp.float32) + p_ref[3],
      0.0)
  out_ref[...] = jnp.dot(
      z, fw2_ref[...], preferred_element_type=jnp.float32) + p_ref[4]


def _tcc(aggs, h1s, dinv, w2, fw1, fw2, params):
  bq = pl.BlockSpec((NB, A2_W), lambda i: (i, 0))
  return pl.pallas_call(
      _tcc_body,
      grid=(GRID,),
      in_specs=[bq, bq, bq, bq, bq, bq, bq, bq,
                pl.BlockSpec((NB, 1), lambda i: (i, 0)),
                pl.BlockSpec((D_H, D_H), lambda i: (0, 0)),
                pl.BlockSpec((D_H, D_H), lambda i: (0, 0)),
                pl.BlockSpec((D_H, D_H), lambda i: (0, 0)),
                pl.BlockSpec((8, D_H), lambda i: (0, 0))],
      out_specs=pl.BlockSpec((NB, D_H), lambda i: (i, 0)),
      out_shape=jax.ShapeDtypeStruct((N, D_H), jnp.float32),
  )(*aggs, *h1s, dinv, w2, fw1, fw2, params)


def kernel(x, edge_index, W1, b1, g1, be1, W2, b2, g2, be2, fw1, fb1, fw2, fb2):
  f32 = jnp.float32
  src = edge_index[0]
  dst = edge_index[1]
  pad = E_PAD - E
  ar = jnp.arange(pad, dtype=jnp.int32)
  srcp = jnp.concatenate([src, ar % N])
  dstp = jnp.concatenate([dst, N + ar % (N_PAD - N)])
  zeros1 = jnp.zeros((N_PAD,), f32)
  zeros8 = jnp.zeros((N_PAD, A1_W), f32)
  zeros16 = jnp.zeros((N_PAD, A2_W), f32)

  x8 = jnp.pad(x, ((0, 0), (0, A1_W - D_IN)))
  w1p = jnp.pad(W1, ((0, A1_W - D_IN), (0, 0)))
  zrow = jnp.zeros((D_H,), f32)
  params1 = jnp.stack([b1, g1, be1, zrow, zrow, zrow, zrow, zrow])
  params2 = jnp.stack([b2, g2, be2, fb1, fb2, zrow, zrow, zrow])

  dg0, dg1 = _deg_call(dstp, zeros1)
  dinv, xs = _tca(dg0.reshape(N_PAD, 1), dg1.reshape(N_PAD, 1), x8)
  a1p0, a1p1 = _agg1_call(srcp, dstp, xs, zeros8)
  h1s = _tcb(a1p0, a1p1, x8, dinv, w1p, params1)
  aggs = list(_agg2_call(srcp, dstp, *h1s, zeros16))
  return _tcc(aggs, h1s, dinv, W2, fw1, fw2, params2)


# speedup vs baseline: 1.0021x; 1.0021x over previous
import jax
import jax.numpy as jnp
from jax import lax
from jax.experimental import pallas as pl
from jax.experimental.pallas import tpu as pltpu
from jax.experimental.pallas import tpu_sc as plsc

N = 100_000
E = 1_600_000
D_IN = 4
D_H = 64

NC = 2
NS = 16
E_PAD = 1_638_400
N_PAD = 102_400
NT_ROWS = N_PAD // NS

def _get_mesh():
  return plsc.VectorSubcoreMesh(
      core_axis_name="c", subcore_axis_name="s", num_cores=NC, num_subcores=NS)


def _zero_acc(zeros_hbm, acc, sid, nt=NT_ROWS):
  pltpu.sync_copy(zeros_hbm.at[pl.ds(sid * nt, nt)],
                  acc.at[pl.ds(sid * nt, nt)])


DSUB = 256
CSUB = 1
STEP_E = DSUB * CSUB


def _deg_body(dst1, zeros1, out0, out1, acc, i0, i1, i2, i3, i4, i5, i6, i7,
              ones, semi, sems):
  cid = lax.axis_index("c")
  sid = lax.axis_index("s")
  ixd = ((i0, i1), (i2, i3), (i4, i5), (i6, i7))
  for k in range(DSUB // 16):
    ones[pl.ds(16 * k, 16)] = jnp.full((16,), 1.0, jnp.float32)
  _zero_acc(zeros1, acc, sid)
  plsc.subcore_barrier()
  ept = E_PAD // (NC * NS)
  base = cid * (E_PAD // 2) + sid * ept
  nstep = ept // STEP_E

  def loadidx(slot, c):
    for j in range(CSUB):
      pltpu.async_copy(dst1.at[pl.ds(base + c * STEP_E + j * DSUB, DSUB)],
                       ixd[slot][j], semi)

  def iwait():
    for j in range(CSUB):
      pltpu.make_async_copy(dst1.at[pl.ds(0, DSUB)], ixd[0][j], semi).wait()

  def fire_s(slot):
    for j in range(CSUB):
      pltpu.async_copy(ones, acc.at[ixd[slot][j]], sems, add=True)

  def sdrain():
    for j in range(CSUB):
      pltpu.make_async_copy(zeros1.at[pl.ds(0, DSUB)], ones, sems).wait()

  loadidx(0, 0)
  loadidx(1, 1)

  def step4(q, carry):
    for k in range(4):
      c = 4 * q + k

      @pl.when(c >= 2)
      def _():
        sdrain()
      iwait()
      fire_s(k)

      @pl.when(c + 2 < nstep)
      def _():
        loadidx((k + 2) % 4, c + 2)
    return carry
  lax.fori_loop(0, nstep // 4, step4, 0)
  sdrain()
  sdrain()
  plsc.subcore_barrier()

  @pl.when(cid == 0)
  def _():
    pltpu.sync_copy(acc.at[pl.ds(sid * NT_ROWS, NT_ROWS)],
                    out0.at[pl.ds(sid * NT_ROWS, NT_ROWS)])

  @pl.when(cid == 1)
  def _():
    pltpu.sync_copy(acc.at[pl.ds(sid * NT_ROWS, NT_ROWS)],
                    out1.at[pl.ds(sid * NT_ROWS, NT_ROWS)])


def _deg_call(dst1, zeros1):
  return pl.kernel(
      _deg_body,
      out_type=[jax.ShapeDtypeStruct((N_PAD,), jnp.float32),
                jax.ShapeDtypeStruct((N_PAD,), jnp.float32)],
      mesh=_get_mesh(),
      compiler_params=pltpu.CompilerParams(use_tc_tiling_on_sc=False),
      scratch_types=(
          [pltpu.VMEM_SHARED((N_PAD,), jnp.float32)]
          + [pltpu.VMEM((DSUB,), jnp.int32) for _ in range(8)]
          + [pltpu.VMEM((DSUB,), jnp.float32),
             pltpu.SemaphoreType.DMA,
             pltpu.SemaphoreType.DMA]
      ),
  )(dst1, zeros1)


A1_W = 8
A2_W = 16
IXR = 8
RBR = 4


def _agg_stream(tbl, acc, src1, dst1, zfull, base, nstep, w,
                ixs, ixd, rbs, semi, semg, sems):
  def loadidx(slot, c):
    for j in range(CSUB):
      off = base + c * STEP_E + j * DSUB
      pltpu.async_copy(src1.at[pl.ds(off, DSUB)], ixs[slot][j], semi)
      pltpu.async_copy(dst1.at[pl.ds(off, DSUB)], ixd[slot][j], semi)

  zrows = zfull.at[pl.ds(0, STEP_E)]
  nrow_iw = 2 * CSUB * DSUB // w

  def iwait():
    pltpu.make_async_copy(zfull.at[pl.ds(0, nrow_iw)],
                          rbs[0].at[pl.ds(0, nrow_iw)], semi).wait()

  def fire_g(slot, rb):
    for j in range(CSUB):
      pltpu.async_copy(tbl.at[ixs[slot][j]],
                       rb.at[pl.ds(j * DSUB, DSUB)], semg)

  def fire_s(slot, rb):
    for j in range(CSUB):
      pltpu.async_copy(rb.at[pl.ds(j * DSUB, DSUB)],
                       acc.at[ixd[slot][j]], sems, add=True)

  def gdrain():
    pltpu.make_async_copy(zrows, rbs[0], semg).wait()

  def sdrain():
    pltpu.make_async_copy(zrows, rbs[0], sems).wait()

  loadidx(0, 0)
  loadidx(1, 1)

  def step8(q, carry):
    for k in range(IXR):
      c = IXR * q + k

      @pl.when(c >= 4)
      def _():
        sdrain()
      iwait()
      fire_g(k, rbs[k % RBR])

      @pl.when(c + 2 < nstep)
      def _():
        loadidx((k + 2) % IXR, c + 2)

      @pl.when(c >= 1)
      def _():
        gdrain()
        fire_s((k - 1) % IXR, rbs[(k - 1) % RBR])
    return carry
  lax.fori_loop(0, nstep // IXR, step8, 0)
  gdrain()
  fire_s((nstep - 1) % IXR, rbs[(nstep - 1) % RBR])
  for _ in range(4):
    sdrain()


def _agg_scratch(w):
  return ([pltpu.VMEM((DSUB,), jnp.int32) for _ in range(2 * CSUB * IXR)]
          + [pltpu.VMEM((STEP_E, w), jnp.float32) for _ in range(RBR)]
          + [pltpu.SemaphoreType.DMA] * 3)


def _slots(refs):
  n = CSUB * IXR
  ixs = tuple(tuple(refs[s * CSUB + j] for j in range(CSUB))
              for s in range(IXR))
  ixd = tuple(tuple(refs[n + s * CSUB + j] for j in range(CSUB))
              for s in range(IXR))
  rbs = tuple(refs[2 * n:2 * n + RBR])
  semi, semg, sems = refs[2 * n + RBR:2 * n + RBR + 3]
  return ixs, ixd, rbs, semi, semg, sems


def _agg1_body(src1, dst1, xs, zeros8, out0, out1, acc, xsp, *refs):
  cid = lax.axis_index("c")
  sid = lax.axis_index("s")
  ixs, ixd, rbs, semi, semg, sems = _slots(refs)
  _zero_acc(zeros8, acc, sid)
  pltpu.sync_copy(xs.at[pl.ds(sid * NT_ROWS, NT_ROWS)],
                  xsp.at[pl.ds(sid * NT_ROWS, NT_ROWS)])
  plsc.subcore_barrier()
  ept = E_PAD // (NC * NS)
  base = cid * (E_PAD // 2) + sid * ept
  _agg_stream(xsp, acc, src1, dst1, zeros8,
              base, ept // STEP_E, A1_W, ixs, ixd, rbs, semi, semg, sems)
  plsc.subcore_barrier()

  @pl.when(cid == 0)
  def _():
    pltpu.sync_copy(acc.at[pl.ds(sid * NT_ROWS, NT_ROWS)],
                    out0.at[pl.ds(sid * NT_ROWS, NT_ROWS)])

  @pl.when(cid == 1)
  def _():
    pltpu.sync_copy(acc.at[pl.ds(sid * NT_ROWS, NT_ROWS)],
                    out1.at[pl.ds(sid * NT_ROWS, NT_ROWS)])


def _agg1_call(src1, dst1, xs, zeros8):
  return pl.kernel(
      _agg1_body,
      out_type=[jax.ShapeDtypeStruct((N_PAD, A1_W), jnp.float32),
                jax.ShapeDtypeStruct((N_PAD, A1_W), jnp.float32)],
      mesh=_get_mesh(),
      compiler_params=pltpu.CompilerParams(use_tc_tiling_on_sc=False),
      scratch_types=(
          [pltpu.VMEM_SHARED((N_PAD, A1_W), jnp.float32),
           pltpu.VMEM_SHARED((N_PAD, A1_W), jnp.float32)]
          + _agg_scratch(A1_W)
      ),
  )(src1, dst1, xs, zeros8)


def _agg2_quarter(src1, dst1, tbl, zeros16, out, acc, slots, sid):
  ixs, ixd, rbs, semi, semg, sems = slots
  _zero_acc(zeros16, acc, sid)
  plsc.subcore_barrier()
  ept = E_PAD // NS
  base = sid * ept
  _agg_stream(tbl, acc, src1, dst1, zeros16,
              base, ept // STEP_E, A2_W, ixs, ixd, rbs, semi, semg, sems)
  plsc.subcore_barrier()
  pltpu.sync_copy(acc.at[pl.ds(sid * NT_ROWS, NT_ROWS)],
                  out.at[pl.ds(sid * NT_ROWS, NT_ROWS)])
  plsc.subcore_barrier()


def _agg2_body(src1, dst1, t0, t1, t2, t3, zeros16, o0, o1, o2, o3, acc,
               *refs):
  cid = lax.axis_index("c")
  sid = lax.axis_index("s")
  slots = _slots(refs)

  @pl.when(cid == 0)
  def _():
    _agg2_quarter(src1, dst1, t0, zeros16, o0, acc, slots, sid)
    _agg2_quarter(src1, dst1, t1, zeros16, o1, acc, slots, sid)

  @pl.when(cid == 1)
  def _():
    _agg2_quarter(src1, dst1, t2, zeros16, o2, acc, slots, sid)
    _agg2_quarter(src1, dst1, t3, zeros16, o3, acc, slots, sid)


def _agg2_call(src1, dst1, t0, t1, t2, t3, zeros16):
  sds = jax.ShapeDtypeStruct((N_PAD, A2_W), jnp.float32)
  return pl.kernel(
      _agg2_body,
      out_type=[sds, sds, sds, sds],
      mesh=_get_mesh(),
      compiler_params=pltpu.CompilerParams(use_tc_tiling_on_sc=False),
      scratch_types=(
          [pltpu.VMEM_SHARED((N_PAD, A2_W), jnp.float32)]
          + _agg_scratch(A2_W)
      ),
  )(src1, dst1, t0, t1, t2, t3, zeros16)


NB = 2000
GRID = N // NB


def _tca_body(dg0_ref, dg1_ref, x8_ref, dinv_ref, xs_ref):
  deg = dg0_ref[...] + dg1_ref[...] + 1.0
  dinv = lax.rsqrt(deg)
  dinv_ref[...] = dinv
  xs_ref[...] = x8_ref[...] * dinv


def _tca(dg0, dg1, x8):
  return pl.pallas_call(
      _tca_body,
      grid=(GRID,),
      in_specs=[
          pl.BlockSpec((NB, 1), lambda i: (i, 0)),
          pl.BlockSpec((NB, 1), lambda i: (i, 0)),
          pl.BlockSpec((NB, A1_W), lambda i: (i, 0)),
      ],
      out_specs=[
          pl.BlockSpec((NB, 1), lambda i: (i, 0)),
          pl.BlockSpec((NB, A1_W), lambda i: (i, 0)),
      ],
      out_shape=[
          jax.ShapeDtypeStruct((N, 1), jnp.float32),
          jax.ShapeDtypeStruct((N_PAD, A1_W), jnp.float32),
      ],
  )(dg0, dg1, x8)


def _layer_norm_relu(h, g, b):
  mu = jnp.mean(h, axis=1, keepdims=True)
  xc = h - mu
  var = jnp.mean(xc * xc, axis=1, keepdims=True)
  return jnp.maximum(xc * lax.rsqrt(var + 1e-5) * g + b, 0.0)


def _tcb_body(aggp0_ref, aggp1_ref, x8_ref, dinv_ref, w1_ref, p_ref,
              o0_ref, o1_ref, o2_ref, o3_ref):
  di = dinv_ref[...]
  agg = aggp0_ref[...] + aggp1_ref[...]
  pre = di * agg + (di * di) * x8_ref[...]
  h = jnp.dot(pre, w1_ref[...], preferred_element_type=jnp.float32)
  h = h + p_ref[0]
  h = _layer_norm_relu(h, p_ref[1], p_ref[2])
  h = h * di
  o0_ref[...] = h[:, 0:16]
  o1_ref[...] = h[:, 16:32]
  o2_ref[...] = h[:, 32:48]
  o3_ref[...] = h[:, 48:64]


def _tcb(aggp0, aggp1, x8, dinv, w1p, params):
  sds = jax.ShapeDtypeStruct((N, A2_W), jnp.float32)
  bs = pl.BlockSpec((NB, A2_W), lambda i: (i, 0))
  return pl.pallas_call(
      _tcb_body,
      grid=(GRID,),
      in_specs=[
          pl.BlockSpec((NB, A1_W), lambda i: (i, 0)),
          pl.BlockSpec((NB, A1_W), lambda i: (i, 0)),
          pl.BlockSpec((NB, A1_W), lambda i: (i, 0)),
          pl.BlockSpec((NB, 1), lambda i: (i, 0)),
          pl.BlockSpec((A1_W, D_H), lambda i: (0, 0)),
          pl.BlockSpec((8, D_H), lambda i: (0, 0)),
      ],
      out_specs=[bs, bs, bs, bs],
      out_shape=[sds, sds, sds, sds],
  )(aggp0, aggp1, x8, dinv, w1p, params)


def _tcc_body(a0, a1, a2, a3, h0, h1, h2, h3, dinv_ref, w2_ref, fw1_ref,
              fw2_ref, p_ref, out_ref):
  di = dinv_ref[...]
  w2 = w2_ref[...]
  z = jnp.zeros((NB, D_H), jnp.float32)
  for k, (a, h) in enumerate(((a0, h0), (a1, h1), (a2, h2), (a3, h3))):
    zk = di * (a[...] + h[...])
    z = z + jnp.dot(zk, w2[16 * k:16 * (k + 1), :],
                    preferred_element_type=jnp.float32)
  z = z + p_ref[0]
  z = _layer_norm_relu(z, p_ref[1], p_ref[2])
  z = jnp.maximum(
      jnp.dot(z, fw1_ref[...], preferred_element_type=jnp.float32) + p_ref[3],
      0.0)
  out_ref[...] = jnp.dot(
      z, fw2_ref[...], preferred_element_type=jnp.float32) + p_ref[4]


def _tcc(aggs, h1s, dinv, w2, fw1, fw2, params):
  bq = pl.BlockSpec((NB, A2_W), lambda i: (i, 0))
  return pl.pallas_call(
      _tcc_body,
      grid=(GRID,),
      in_specs=[bq, bq, bq, bq, bq, bq, bq, bq,
                pl.BlockSpec((NB, 1), lambda i: (i, 0)),
                pl.BlockSpec((D_H, D_H), lambda i: (0, 0)),
                pl.BlockSpec((D_H, D_H), lambda i: (0, 0)),
                pl.BlockSpec((D_H, D_H), lambda i: (0, 0)),
                pl.BlockSpec((8, D_H), lambda i: (0, 0))],
      out_specs=pl.BlockSpec((NB, D_H), lambda i: (i, 0)),
      out_shape=jax.ShapeDtypeStruct((N, D_H), jnp.float32),
  )(*aggs, *h1s, dinv, w2, fw1, fw2, params)


def kernel(x, edge_index, W1, b1, g1, be1, W2, b2, g2, be2, fw1, fb1, fw2, fb2):
  f32 = jnp.float32
  src = edge_index[0]
  dst = edge_index[1]
  pad = E_PAD - E
  ar = jnp.arange(pad, dtype=jnp.int32)
  srcp = jnp.concatenate([src, ar % N])
  dstp = jnp.concatenate([dst, N + ar % (N_PAD - N)])
  zeros1 = jnp.zeros((N_PAD,), f32)
  zeros8 = jnp.zeros((N_PAD, A1_W), f32)
  zeros16 = jnp.zeros((N_PAD, A2_W), f32)

  x8 = jnp.pad(x, ((0, 0), (0, A1_W - D_IN)))
  w1p = jnp.pad(W1, ((0, A1_W - D_IN), (0, 0)))
  zrow = jnp.zeros((D_H,), f32)
  params1 = jnp.stack([b1, g1, be1, zrow, zrow, zrow, zrow, zrow])
  params2 = jnp.stack([b2, g2, be2, fb1, fb2, zrow, zrow, zrow])

  dg0, dg1 = _deg_call(dstp, zeros1)
  dinv, xs = _tca(dg0.reshape(N_PAD, 1), dg1.reshape(N_PAD, 1), x8)
  a1p0, a1p1 = _agg1_call(srcp, dstp, xs, zeros8)
  h1s = _tcb(a1p0, a1p1, x8, dinv, w1p, params1)
  aggs = list(_agg2_call(srcp, dstp, *h1s, zeros16))
  return _tcc(aggs, h1s, dinv, W2, fw1, fw2, params2)
